# baseline (speedup 1.0000x reference)
import functools

import jax
import jax.numpy as jnp
from jax.experimental import pallas as pl
from jax.experimental.pallas import tpu as pltpu

N = 2048
_DOT = functools.partial(jnp.dot, preferred_element_type=jnp.float32)
_BF = jnp.bfloat16

_CH = 256
_NCH = N // _CH
_SR = 128


def _dot_nt(a, b):
    return jax.lax.dot_general(a, b, (((1,), (1,)), ((), ())),
                               preferred_element_type=jnp.float32)


def _gcn3(x, a_ref, w1, w2, w3):
    u = _DOT(x, w1.astype(_BF)).astype(_BF)
    o = jnp.maximum(_DOT(a_ref[:], u), 0.0).astype(_BF)
    o = jnp.maximum(_DOT(a_ref[:], _DOT(o, w2.astype(_BF)).astype(_BF)), 0.0).astype(_BF)
    return jnp.maximum(_DOT(a_ref[:], _DOT(o, w3.astype(_BF)).astype(_BF)), 0.0)


def _body(x_ref, xm_ref, a1_ref, a2_ref, g_ref,
          wge1_ref, wge2_ref, wge3_ref, wgd1_ref, wgd2_ref, wgd3_ref,
          whe1_ref, whe2_ref, whe3_ref, whd1_ref, whd2_ref, whd3_ref,
          alpha_ref,
          h_ref, s1_ref, s2_ref, s3_ref, x1_ref, x2_ref, x3_ref,
          a1s_ref, a2s_ref, gs_ref, stage_ref, sems,
          hpack_ref, x1s_ref, x2s_ref, x3s_ref,
          tile_ref, tsems):
    plan = [(a1s_ref, a1_ref, c) for c in range(_NCH)] \
         + [(a2s_ref, a2_ref, c) for c in range(_NCH)] \
         + [(gs_ref, g_ref, c) for c in range(_NCH)]

    def stream_copy(t):
        _, src, c = plan[t]
        return pltpu.make_async_copy(
            src.at[pl.ds(c * _CH, _CH), :],
            stage_ref.at[t % 2],
            sems.at[t % 2],
        )

    stream_copy(0).start()
    for t in range(len(plan)):
        if t + 1 < len(plan):
            stream_copy(t + 1).start()
        dst, _, c = plan[t]
        stream_copy(t).wait()
        dst[pl.ds(c * _CH, _CH), :] = stage_ref[t % 2].astype(_BF)

    h1 = _gcn3(x_ref[:], a1s_ref, wge1_ref[:], wge2_ref[:], wge3_ref[:])
    h2 = _gcn3(xm_ref[:], a2s_ref, wge1_ref[:], wge2_ref[:], wge3_ref[:])
    h3 = _gcn3(x_ref[:], gs_ref, whe1_ref[:], whe2_ref[:], whe3_ref[:])
    alpha = alpha_ref[0, 0]
    h = alpha * 0.5 * (h1 + h2) + (1.0 - alpha) * h3
    h_ref[:] = h
    hpack_ref[:, 0:32] = h1.astype(_BF)
    hpack_ref[:, 32:64] = h2.astype(_BF)
    hpack_ref[:, 64:96] = h3.astype(_BF)

    h_bf = h.astype(_BF)
    x1 = _gcn3(h_bf, a1s_ref, wgd1_ref[:], wgd2_ref[:], wgd3_ref[:])
    x2 = _gcn3(h_bf, a2s_ref, wgd1_ref[:], wgd2_ref[:], wgd3_ref[:])
    x3 = _gcn3(h_bf, gs_ref, whd1_ref[:], whd2_ref[:], whd3_ref[:])
    x1_ref[:] = x1
    x2_ref[:] = x2
    x3_ref[:] = x3
    x1s_ref[:] = x1.astype(_BF)
    x2s_ref[:] = x2.astype(_BF)
    x3s_ref[:] = x3.astype(_BF)

    pend = [None, None]
    idx = 0
    for k, (xs, out) in enumerate(((x1s_ref, s1_ref),
                                   (x2s_ref, s2_ref),
                                   (x3s_ref, s3_ref))):
        hj = hpack_ref[:, k * 32:(k + 1) * 32]
        for i in range(N // _SR):
            slot = idx % 2
            if pend[slot] is not None:
                pend[slot].wait()
            hi = hpack_ref[pl.ds(i * _SR, _SR), k * 32:(k + 1) * 32]
            xi = xs[pl.ds(i * _SR, _SR), :]
            t_enc = jnp.tanh(0.5 * _dot_nt(hi, hj))
            t_dec = jnp.tanh(0.5 * _dot_nt(xi, xs[:]))
            tile_ref[slot] = 0.5 + 0.25 * (t_enc + t_dec)
            cp = pltpu.make_async_copy(
                tile_ref.at[slot],
                out.at[pl.ds(i * _SR, _SR), :],
                tsems.at[slot],
            )
            cp.start()
            pend[slot] = cp
            idx += 1
    for cp in pend:
        if cp is not None:
            cp.wait()


def kernel(x, x_mask, A1, A2, G, Wg_e1, Wg_e2, Wg_e3, Wg_d1, Wg_d2, Wg_d3,
           Wh_e1, Wh_e2, Wh_e3, Wh_d1, Wh_d2, Wh_d3, alpha):
    f32 = jnp.float32
    vspec = pl.BlockSpec(memory_space=pltpu.MemorySpace.VMEM)
    aspec = pl.BlockSpec(memory_space=pltpu.MemorySpace.HBM)
    out_shapes = (
        jax.ShapeDtypeStruct((N, 32), f32),
        jax.ShapeDtypeStruct((N, N), f32),
        jax.ShapeDtypeStruct((N, N), f32),
        jax.ShapeDtypeStruct((N, N), f32),
        jax.ShapeDtypeStruct((N, 256), f32),
        jax.ShapeDtypeStruct((N, 256), f32),
        jax.ShapeDtypeStruct((N, 256), f32),
    )
    out_specs = (vspec, aspec, aspec, aspec, vspec, vspec, vspec)
    scratch = [
        pltpu.VMEM((N, N), _BF),
        pltpu.VMEM((N, N), _BF),
        pltpu.VMEM((N, N), _BF),
        pltpu.VMEM((2, _CH, N), f32),
        pltpu.SemaphoreType.DMA((2,)),
        pltpu.VMEM((N, 96), _BF),
        pltpu.VMEM((N, 256), _BF),
        pltpu.VMEM((N, 256), _BF),
        pltpu.VMEM((N, 256), _BF),
        pltpu.VMEM((2, _SR, N), f32),
        pltpu.SemaphoreType.DMA((2,)),
    ]
    return pl.pallas_call(
        _body,
        in_specs=[vspec, vspec, aspec, aspec, aspec] + [vspec] * 13,
        out_specs=out_specs,
        out_shape=out_shapes,
        scratch_shapes=scratch,
        compiler_params=pltpu.CompilerParams(
            vmem_limit_bytes=100 * 1024 * 1024),
    )(x.astype(_BF), x_mask.astype(_BF), A1, A2, G,
      Wg_e1, Wg_e2, Wg_e3, Wg_d1, Wg_d2, Wg_d3,
      Wh_e1, Wh_e2, Wh_e3, Wh_d1, Wh_d2, Wh_d3, alpha.reshape(1, 1))
